# baseline (speedup 1.0000x reference)
import functools

import jax
import jax.numpy as jnp
from jax import lax
from jax.experimental import pallas as pl
from jax.experimental.pallas import tpu as pltpu
from jax.experimental.pallas import tpu_sc as plsc

ROWS, COLS = 16384, 128

_info = plsc.get_sparse_core_info()
_NC, _NS, _L = _info.num_cores, _info.num_subcores, _info.num_lanes
_NW = _NC * _NS
_RPW = ROWS // _NW

_mesh = plsc.VectorSubcoreMesh(core_axis_name="c", subcore_axis_name="s")

_NPIECE = 4
_RPP = _RPW // _NPIECE


@functools.partial(
    pl.kernel,
    mesh=_mesh,
    out_type=jax.ShapeDtypeStruct((ROWS, COLS), jnp.float32),
    scratch_types=[
        pltpu.VMEM((_RPW, COLS), jnp.float32),
        *([pltpu.SemaphoreType.DMA] * _NPIECE),
        *([pltpu.SemaphoreType.DMA] * _NPIECE),
    ],
)
def _copy_assign(x_hbm, out_hbm, buf, *sems):
    in_sems, out_sems = sems[:_NPIECE], sems[_NPIECE:]
    wid = lax.axis_index("s") * _NC + lax.axis_index("c")
    base = wid * _RPW

    gathers = []
    for k in range(_NPIECE):
        gathers.append(pltpu.async_copy(
            x_hbm.at[pl.ds(base + k * _RPP, _RPP)],
            buf.at[pl.ds(k * _RPP, _RPP)],
            in_sems[k]))

    scatters = []
    for k in range(_NPIECE):
        gathers[k].wait()
        if k == 0:
            @pl.when(wid == 0)
            def _patch():
                v = buf[1, pl.ds(0, _L)]
                buf[1, pl.ds(0, _L)] = jnp.where(
                    lax.iota(jnp.int32, _L) == 2, jnp.float32(1.0), v)
        scatters.append(pltpu.async_copy(
            buf.at[pl.ds(k * _RPP, _RPP)],
            out_hbm.at[pl.ds(base + k * _RPP, _RPP)],
            out_sems[k]))
    for s in scatters:
        s.wait()


def kernel(x):
    return _copy_assign(x)


# speedup vs baseline: 1.0066x; 1.0066x over previous
import functools

import jax
import jax.numpy as jnp
from jax import lax
from jax.experimental import pallas as pl
from jax.experimental.pallas import tpu as pltpu
from jax.experimental.pallas import tpu_sc as plsc

ROWS, COLS = 16384, 128

_info = plsc.get_sparse_core_info()
_NC, _NS, _L = _info.num_cores, _info.num_subcores, _info.num_lanes
_NW = _NC * _NS
_RPW = ROWS // _NW

_mesh = plsc.VectorSubcoreMesh(core_axis_name="c", subcore_axis_name="s")

_NPIECE = 2
_RPP = _RPW // _NPIECE


@functools.partial(
    pl.kernel,
    mesh=_mesh,
    out_type=jax.ShapeDtypeStruct((ROWS, COLS), jnp.float32),
    scratch_types=[
        pltpu.VMEM((_RPW, COLS), jnp.float32),
        *([pltpu.SemaphoreType.DMA] * _NPIECE),
        *([pltpu.SemaphoreType.DMA] * _NPIECE),
    ],
)
def _copy_assign(x_hbm, out_hbm, buf, *sems):
    in_sems, out_sems = sems[:_NPIECE], sems[_NPIECE:]
    wid = lax.axis_index("s") * _NC + lax.axis_index("c")
    base = wid * _RPW

    gathers = []
    for k in range(_NPIECE):
        gathers.append(pltpu.async_copy(
            x_hbm.at[pl.ds(base + k * _RPP, _RPP)],
            buf.at[pl.ds(k * _RPP, _RPP)],
            in_sems[k]))

    scatters = []
    for k in range(_NPIECE):
        gathers[k].wait()
        if k == 0:
            @pl.when(wid == 0)
            def _patch():
                v = buf[1, pl.ds(0, _L)]
                buf[1, pl.ds(0, _L)] = jnp.where(
                    lax.iota(jnp.int32, _L) == 2, jnp.float32(1.0), v)
        scatters.append(pltpu.async_copy(
            buf.at[pl.ds(k * _RPP, _RPP)],
            out_hbm.at[pl.ds(base + k * _RPP, _RPP)],
            out_sems[k]))
    for s in scatters:
        s.wait()


def kernel(x):
    return _copy_assign(x)
